# Initial kernel scaffold; baseline (speedup 1.0000x reference)
import jax
import jax.numpy as jnp
from jax.experimental import pallas as pl


def kernel(pred, target, mask):
    raise NotImplementedError("write your pallas kernel here")


# speedup vs baseline: 1.1000x; 1.1000x over previous
import jax
import jax.numpy as jnp
from jax.experimental import pallas as pl
from jax.experimental.pallas import tpu as pltpu

_ROWS = 16384
_COLS = 4096
_BLOCK_ROWS = 512


def _body(pred_ref, target_ref, mask_ref, out_ref, sum_ref, cnt_ref):
    i = pl.program_id(0)

    @pl.when(i == 0)
    def _init():
        sum_ref[0] = jnp.float32(0.0)
        cnt_ref[0] = jnp.int32(0)

    p = pred_ref[...]
    t = target_ref[...]
    m = mask_ref[...]
    valid = jnp.logical_not(jnp.isnan(p)) & jnp.logical_not(jnp.isnan(t)) & jnp.logical_not(m)
    d = jnp.where(valid, p - t, jnp.float32(0.0))
    sum_ref[0] += jnp.sum(d * d)
    cnt_ref[0] += jnp.sum(valid.astype(jnp.int32))

    @pl.when(i == pl.num_programs(0) - 1)
    def _fini():
        out_ref[0, 0] = sum_ref[0] / cnt_ref[0].astype(jnp.float32)


def kernel(pred, target, mask):
    p = pred.reshape(_ROWS, _COLS)
    t = target.reshape(_ROWS, _COLS)
    m = mask.reshape(_ROWS, _COLS)
    grid = (_ROWS // _BLOCK_ROWS,)
    out = pl.pallas_call(
        _body,
        grid=grid,
        in_specs=[
            pl.BlockSpec((_BLOCK_ROWS, _COLS), lambda i: (i, 0)),
            pl.BlockSpec((_BLOCK_ROWS, _COLS), lambda i: (i, 0)),
            pl.BlockSpec((_BLOCK_ROWS, _COLS), lambda i: (i, 0)),
        ],
        out_specs=pl.BlockSpec(memory_space=pltpu.SMEM),
        out_shape=jax.ShapeDtypeStruct((1, 1), jnp.float32),
        scratch_shapes=[
            pltpu.SMEM((1,), jnp.float32),
            pltpu.SMEM((1,), jnp.int32),
        ],
        compiler_params=pltpu.CompilerParams(
            dimension_semantics=("arbitrary",),
        ),
    )(p, t, m)
    return out.reshape(())
